# Initial kernel scaffold; baseline (speedup 1.0000x reference)
import jax
import jax.numpy as jnp
from jax.experimental import pallas as pl


def kernel(x, edge_index):
    raise NotImplementedError("write your pallas kernel here")


# speedup vs baseline: 5.3717x; 5.3717x over previous
import functools

import jax
import jax.numpy as jnp
from jax import lax
from jax.experimental import pallas as pl
from jax.experimental.pallas import tpu as pltpu
from jax.experimental.pallas import tpu_sc as plsc

_N = 10000
_D = 128
_E = 320000
_NC = 2
_NS = 16
_NW = _NC * _NS
_NP = 10112
_RPT = _NP // _NS
_C = 128
_EP = 327680
_CPW = _EP // (_C * _NW)

_PROP_STEP = 8
_ALP = 0.5

_mesh = plsc.VectorSubcoreMesh(core_axis_name="c", subcore_axis_name="s")


def _zero_vmem_2d(ref, rows, cols):
    zv = jnp.zeros((16,), jnp.float32)

    def body(k, _):
        r = k // (cols // 16)
        col = (k % (cols // 16)) * 16
        ref[r, pl.ds(col, 16)] = zv
        return 0

    lax.fori_loop(0, rows * (cols // 16), body, 0)


@functools.partial(
    pl.kernel,
    out_type=(
        jax.ShapeDtypeStruct((_NP, _D), jnp.float32),
        jax.ShapeDtypeStruct((_NP, _D), jnp.float32),
    ),
    mesh=_mesh,
    scratch_types=[
        pltpu.VMEM((_C,), jnp.int32),
        pltpu.VMEM((_C,), jnp.int32),
        pltpu.VMEM((_C, _D), jnp.float32),
        pltpu.VMEM((_C, _D), jnp.float32),
        pltpu.VMEM_SHARED((_NP, _D), jnp.float32),
        pltpu.SemaphoreType.DMA,
    ],
)
def _spmm_step(z_hbm, src_hbm, dst_hbm, p0_hbm, p1_hbm,
               idx_s, idx_d, rows, zbuf, acc, sem):
    c = lax.axis_index("c")
    s = lax.axis_index("s")
    w = c * _NS + s

    _zero_vmem_2d(zbuf, _C, _D)
    base_row = s * _RPT
    for k in range(_RPT // _C):
        pltpu.sync_copy(zbuf, acc.at[pl.ds(base_row + k * _C, _C)])
    rem = _RPT % _C
    if rem:
        pltpu.sync_copy(zbuf.at[pl.ds(0, rem)],
                        acc.at[pl.ds(base_row + (_RPT // _C) * _C, rem)])
    plsc.subcore_barrier()

    cbase = w * _CPW

    def chunk(g, _):
        row = cbase + g
        pltpu.sync_copy(src_hbm.at[row], idx_s)
        pltpu.sync_copy(dst_hbm.at[row], idx_d)
        pltpu.async_copy(z_hbm.at[idx_s], rows, sem).wait()
        pltpu.sync_copy(rows, acc.at[idx_d], add=True)
        return 0

    lax.fori_loop(0, _CPW, chunk, 0)
    plsc.subcore_barrier()

    @pl.when(c == 0)
    def _():
        pltpu.sync_copy(acc.at[pl.ds(base_row, _RPT)],
                        p0_hbm.at[pl.ds(base_row, _RPT)])

    @pl.when(c == 1)
    def _():
        pltpu.sync_copy(acc.at[pl.ds(base_row, _RPT)],
                        p1_hbm.at[pl.ds(base_row, _RPT)])


@functools.partial(
    pl.kernel,
    out_type=(
        jax.ShapeDtypeStruct((_NP, 16), jnp.float32),
        jax.ShapeDtypeStruct((_NP, 16), jnp.float32),
    ),
    mesh=_mesh,
    scratch_types=[
        pltpu.VMEM((_C,), jnp.int32),
        pltpu.VMEM((_C, 16), jnp.float32),
        pltpu.VMEM((_C, 16), jnp.float32),
        pltpu.VMEM_SHARED((_NP, 16), jnp.float32),
    ],
)
def _degree(dst_hbm, d0_hbm, d1_hbm, idx_d, ones, zbuf, acc):
    c = lax.axis_index("c")
    s = lax.axis_index("s")
    w = c * _NS + s

    ov = jnp.ones((16,), jnp.float32)

    def fill(k, _):
        ones[k, pl.ds(0, 16)] = ov
        return 0

    lax.fori_loop(0, _C, fill, 0)
    _zero_vmem_2d(zbuf, _C, 16)

    base_row = s * _RPT
    for k in range(_RPT // _C):
        pltpu.sync_copy(zbuf, acc.at[pl.ds(base_row + k * _C, _C)])
    rem = _RPT % _C
    if rem:
        pltpu.sync_copy(zbuf.at[pl.ds(0, rem)],
                        acc.at[pl.ds(base_row + (_RPT // _C) * _C, rem)])
    plsc.subcore_barrier()

    cbase = w * _CPW

    def chunk(g, _):
        pltpu.sync_copy(dst_hbm.at[cbase + g], idx_d)
        pltpu.sync_copy(ones, acc.at[idx_d], add=True)
        return 0

    lax.fori_loop(0, _CPW, chunk, 0)
    plsc.subcore_barrier()

    @pl.when(c == 0)
    def _():
        pltpu.sync_copy(acc.at[pl.ds(base_row, _RPT)],
                        d0_hbm.at[pl.ds(base_row, _RPT)])

    @pl.when(c == 1)
    def _():
        pltpu.sync_copy(acc.at[pl.ds(base_row, _RPT)],
                        d1_hbm.at[pl.ds(base_row, _RPT)])


def kernel(x, edge_index):
    src = edge_index[0].astype(jnp.int32)
    dst = edge_index[1].astype(jnp.int32)

    npad = _EP - _E
    pad_src = (jnp.arange(npad, dtype=jnp.int32) * 7919) % _N
    pad_dst = _N + (jnp.arange(npad, dtype=jnp.int32) % (_NP - _N))
    src_p = jnp.concatenate([src, pad_src]).reshape(_NW * _CPW, _C)
    dst_p = jnp.concatenate([dst, pad_dst]).reshape(_NW * _CPW, _C)

    d0, d1 = _degree(dst_p)
    deg = (d0 + d1)[:, 0]

    valid = jnp.arange(_NP) < _N
    inv = jnp.where(deg > 0, 1.0 / jnp.where(deg > 0, deg, 1.0), jnp.inf)
    rsq = jnp.where(valid & (deg > 0), lax.rsqrt(jnp.where(deg > 0, deg, 1.0)), 0.0)
    c1 = jnp.where(valid, 1.0 - _ALP * inv - _ALP, 0.0)[:, None]
    c2 = jnp.where(valid, _ALP * inv, 0.0)[:, None]
    rsq = rsq[:, None]

    y0 = jnp.pad(x, ((0, _NP - _N), (0, 0)))
    y = y0
    z = rsq * y
    for _ in range(_PROP_STEP):
        p0, p1 = _spmm_step(z, src_p, dst_p)
        y = c1 * y + c2 * y0 + (_ALP * rsq) * (p0 + p1)
        z = rsq * y
    return y[:_N]


# speedup vs baseline: 10.4008x; 1.9362x over previous
import functools

import jax
import jax.numpy as jnp
from jax import lax
from jax.experimental import pallas as pl
from jax.experimental.pallas import tpu as pltpu
from jax.experimental.pallas import tpu_sc as plsc

_N = 10000
_D = 128
_E = 320000
_NC = 2
_NS = 16
_NW = _NC * _NS
_NP = 10112
_RPT = _NP // _NS
_C = 128
_EP = 327680
_CPW = _EP // (_C * _NW)
_BC = 16

_PROP_STEP = 8
_ALP = 0.5

_mesh = plsc.VectorSubcoreMesh(core_axis_name="c", subcore_axis_name="s")


def _zero_acc_slice(zrows_hbm, zbuf, acc, base_row):
    pltpu.sync_copy(zrows_hbm, zbuf)
    for k in range(_RPT // _C):
        pltpu.sync_copy(zbuf, acc.at[pl.ds(base_row + k * _C, _C)])
    rem = _RPT % _C
    if rem:
        pltpu.sync_copy(zbuf.at[pl.ds(0, rem)],
                        acc.at[pl.ds(base_row + (_RPT // _C) * _C, rem)])


@functools.partial(
    pl.kernel,
    out_type=(
        jax.ShapeDtypeStruct((_NP, _D), jnp.float32),
        jax.ShapeDtypeStruct((_NP, _D), jnp.float32),
    ),
    mesh=_mesh,
    scratch_types=[
        pltpu.VMEM((_BC, _C), jnp.int32),
        pltpu.VMEM((_BC, _C), jnp.int32),
        pltpu.VMEM((_C, _D), jnp.float32),
        pltpu.VMEM((_C, _D), jnp.float32),
        pltpu.VMEM_SHARED((_NP, _D), jnp.float32),
        pltpu.SemaphoreType.DMA,
        pltpu.SemaphoreType.DMA,
    ],
)
def _spmm_step(z_hbm, src_hbm, dst_hbm, zrows_hbm, p0_hbm, p1_hbm,
               sidx, didx, rows0, rows1, acc, sem0, sem1):
    c = lax.axis_index("c")
    s = lax.axis_index("s")
    w = c * _NS + s
    base_row = s * _RPT

    _zero_acc_slice(zrows_hbm, rows0, acc, base_row)
    plsc.subcore_barrier()

    for b in range(_CPW // _BC):
        pltpu.sync_copy(src_hbm.at[pl.ds(w * _CPW + b * _BC, _BC)], sidx)
        pltpu.sync_copy(dst_hbm.at[pl.ds(w * _CPW + b * _BC, _BC)], didx)
        pltpu.async_copy(z_hbm.at[sidx.at[0]], rows0, sem0)

        def body(h, _):
            g0 = 2 * h
            pltpu.async_copy(z_hbm.at[sidx.at[g0 + 1]], rows1, sem1)
            pltpu.make_async_copy(z_hbm.at[sidx.at[g0]], rows0, sem0).wait()
            pltpu.sync_copy(rows0, acc.at[didx.at[g0]], add=True)

            @pl.when(h < _BC // 2 - 1)
            def _():
                pltpu.async_copy(z_hbm.at[sidx.at[g0 + 2]], rows0, sem0)

            pltpu.make_async_copy(z_hbm.at[sidx.at[g0 + 1]], rows1, sem1).wait()
            pltpu.sync_copy(rows1, acc.at[didx.at[g0 + 1]], add=True)
            return 0

        lax.fori_loop(0, _BC // 2, body, 0)
    plsc.subcore_barrier()

    @pl.when(c == 0)
    def _():
        pltpu.sync_copy(acc.at[pl.ds(base_row, _RPT)],
                        p0_hbm.at[pl.ds(base_row, _RPT)])

    @pl.when(c == 1)
    def _():
        pltpu.sync_copy(acc.at[pl.ds(base_row, _RPT)],
                        p1_hbm.at[pl.ds(base_row, _RPT)])


@functools.partial(
    pl.kernel,
    out_type=(
        jax.ShapeDtypeStruct((_NP, 16), jnp.float32),
        jax.ShapeDtypeStruct((_NP, 16), jnp.float32),
    ),
    mesh=_mesh,
    scratch_types=[
        pltpu.VMEM((_CPW, _C), jnp.int32),
        pltpu.VMEM((_C, 16), jnp.float32),
        pltpu.VMEM((_C, 16), jnp.float32),
        pltpu.VMEM_SHARED((_NP, 16), jnp.float32),
    ],
)
def _degree(dst_hbm, d0_hbm, d1_hbm, didx, ones, zbuf, acc):
    c = lax.axis_index("c")
    s = lax.axis_index("s")
    w = c * _NS + s
    base_row = s * _RPT

    pltpu.sync_copy(dst_hbm.at[pl.ds(w * _CPW, _CPW)], didx)

    ov = jnp.ones((16,), jnp.float32)
    zv = jnp.zeros((16,), jnp.float32)

    def fill(k, _):
        ones[k, pl.ds(0, 16)] = ov
        zbuf[k, pl.ds(0, 16)] = zv
        return 0

    lax.fori_loop(0, _C, fill, 0)

    for k in range(_RPT // _C):
        pltpu.sync_copy(zbuf, acc.at[pl.ds(base_row + k * _C, _C)])
    rem = _RPT % _C
    if rem:
        pltpu.sync_copy(zbuf.at[pl.ds(0, rem)],
                        acc.at[pl.ds(base_row + (_RPT // _C) * _C, rem)])
    plsc.subcore_barrier()

    def chunk(g, _):
        pltpu.sync_copy(ones, acc.at[didx.at[g]], add=True)
        return 0

    lax.fori_loop(0, _CPW, chunk, 0)
    plsc.subcore_barrier()

    @pl.when(c == 0)
    def _():
        pltpu.sync_copy(acc.at[pl.ds(base_row, _RPT)],
                        d0_hbm.at[pl.ds(base_row, _RPT)])

    @pl.when(c == 1)
    def _():
        pltpu.sync_copy(acc.at[pl.ds(base_row, _RPT)],
                        d1_hbm.at[pl.ds(base_row, _RPT)])


def kernel(x, edge_index):
    src = edge_index[0].astype(jnp.int32)
    dst = edge_index[1].astype(jnp.int32)

    npad = _EP - _E
    pad_src = (jnp.arange(npad, dtype=jnp.int32) * 7919) % _N
    pad_dst = _N + (jnp.arange(npad, dtype=jnp.int32) % (_NP - _N))
    src_p = jnp.concatenate([src, pad_src]).reshape(_NW * _CPW, _C)
    dst_p = jnp.concatenate([dst, pad_dst]).reshape(_NW * _CPW, _C)

    d0, d1 = _degree(dst_p)
    deg = (d0 + d1)[:, 0]

    valid = jnp.arange(_NP) < _N
    inv = jnp.where(deg > 0, 1.0 / jnp.where(deg > 0, deg, 1.0), jnp.inf)
    rsq = jnp.where(valid & (deg > 0), lax.rsqrt(jnp.where(deg > 0, deg, 1.0)), 0.0)
    c1 = jnp.where(valid, 1.0 - _ALP * inv - _ALP, 0.0)[:, None]
    c2 = jnp.where(valid, _ALP * inv, 0.0)[:, None]
    rsq = rsq[:, None]

    zrows = jnp.zeros((_C, _D), jnp.float32)
    y0 = jnp.pad(x, ((0, _NP - _N), (0, 0)))
    y = y0
    z = rsq * y
    for _ in range(_PROP_STEP):
        p0, p1 = _spmm_step(z, src_p, dst_p, zrows)
        y = c1 * y + c2 * y0 + (_ALP * rsq) * (p0 + p1)
        z = rsq * y
    return y[:_N]
